# baseline (speedup 1.0000x reference)
import jax
import jax.numpy as jnp
from jax.experimental import pallas as pl
from jax.experimental.pallas import tpu as pltpu

_N = 4096
_D = 512
_NUM_CLASSES = 64
_MARGIN = 0.5
_BLK = 512
_NB = _N // _BLK
_STEPS = _NB * (_NB + 1) // 2
_KAUG = _D + 4 + _NUM_CLASSES
_KPAD = 640
_TWO_B = 16384.0


def _triplet_kernel(e_ref, lab_ref, out_ref,
                    lhs_s, rhs_s, rp_s, rn_s, cp_s, cn_s):
    t = pl.program_id(0)
    f = jnp.float32

    i = jnp.int32(0)
    for k in range(1, _NB):
        off_k = k * _NB - k * (k - 1) // 2
        i = i + (t >= off_k).astype(jnp.int32)
    off_i = i * _NB - i * (i - 1) // 2
    j = t - off_i + i

    @pl.when(t == 0)
    def _prologue():
        e = e_ref[...]
        oh = jnp.where(
            lab_ref[...] == jax.lax.broadcasted_iota(
                jnp.int32, (_N, _NUM_CLASSES), 1),
            jnp.float32(128.0), jnp.float32(0.0))
        sq = jnp.sum(e * e, axis=1, keepdims=True)
        hi = sq.astype(jnp.bfloat16).astype(f)
        lo = sq - hi
        ones = jnp.ones((_N, 1), f)
        zpad = jnp.zeros((_N, _KPAD - _KAUG), f)
        lhs_s[...] = jnp.concatenate(
            [-2.0 * e, hi, lo, ones, ones, oh, zpad], axis=1
        ).astype(jnp.bfloat16)
        rhs_s[...] = jnp.concatenate(
            [e, ones, ones, hi, lo, oh, zpad], axis=1
        ).astype(jnp.bfloat16)
        ninf = jnp.full((_NB, _BLK), -jnp.inf, f)
        rp_s[...] = ninf
        cp_s[...] = ninf
        rn_s[...] = -ninf
        cn_s[...] = -ninf

    dims = (((1,), (1,)), ((), ()))
    lhs = lhs_s[pl.ds(i * _BLK, _BLK), :]
    rhs = rhs_s[pl.ds(j * _BLK, _BLK), :]
    tb = jax.lax.dot_general(lhs, rhs, dims, preferred_element_type=f)

    row_max = jnp.max(tb, axis=1, keepdims=True)
    row_min = jnp.min(tb, axis=1, keepdims=True)
    rp_blk = rp_s[pl.ds(i, 1), :]
    rn_blk = rn_s[pl.ds(i, 1), :]
    rp_s[pl.ds(i, 1), :] = jnp.maximum(rp_blk, row_max.reshape(1, _BLK))
    rn_s[pl.ds(i, 1), :] = jnp.minimum(rn_blk, row_min.reshape(1, _BLK))

    col_max = jnp.max(tb, axis=0, keepdims=True)
    col_min = jnp.min(tb, axis=0, keepdims=True)
    cp_s[pl.ds(j, 1), :] = jnp.maximum(cp_s[pl.ds(j, 1), :], col_max)
    cn_s[pl.ds(j, 1), :] = jnp.minimum(cn_s[pl.ds(j, 1), :], col_min)

    @pl.when(t == _STEPS - 1)
    def _epilogue():
        pos = jnp.maximum(rp_s[...], cp_s[...]) - _TWO_B
        neg = jnp.minimum(rn_s[...], cn_s[...])
        out_ref[...] = jnp.sum(
            jnp.maximum(pos - neg + _MARGIN, 0.0), keepdims=True
        ).reshape(1, 1)


def kernel(embeds, labels):
    total = pl.pallas_call(
        _triplet_kernel,
        grid=(_STEPS,),
        in_specs=[
            pl.BlockSpec((_N, _D), lambda t: (0, 0)),
            pl.BlockSpec((_N, 1), lambda t: (0, 0)),
        ],
        out_specs=pl.BlockSpec((1, 1), lambda t: (0, 0)),
        out_shape=jax.ShapeDtypeStruct((1, 1), jnp.float32),
        scratch_shapes=[
            pltpu.VMEM((_N, _KPAD), jnp.bfloat16),
            pltpu.VMEM((_N, _KPAD), jnp.bfloat16),
            pltpu.VMEM((_NB, _BLK), jnp.float32),
            pltpu.VMEM((_NB, _BLK), jnp.float32),
            pltpu.VMEM((_NB, _BLK), jnp.float32),
            pltpu.VMEM((_NB, _BLK), jnp.float32),
        ],
        compiler_params=pltpu.CompilerParams(
            dimension_semantics=("arbitrary",),
        ),
    )(embeds, labels.reshape(_N, 1))

    return total[0, 0] / _N


# speedup vs baseline: 1.7224x; 1.7224x over previous
import functools

import jax
import jax.numpy as jnp
from jax.experimental import pallas as pl
from jax.experimental.pallas import tpu as pltpu

_N = 4096
_D = 512
_NUM_CLASSES = 64
_MARGIN = 0.5
_BLK = 1024
_NB = _N // _BLK
_STEPS = _NB * (_NB + 1) // 2
_KAUG = _D + 4 + _NUM_CLASSES
_KPAD = 640
_TWO_B = 16384.0
_LANES = 128


def _triplet_kernel(e_ref, lab_ref, out_ref,
                    lhs_s, rhs_s, rp_s, rn_s, cp_s, cn_s):
    t = pl.program_id(0)
    f = jnp.float32

    i = jnp.int32(0)
    for k in range(1, _NB):
        off_k = k * _NB - k * (k - 1) // 2
        i = i + (t >= off_k).astype(jnp.int32)
    off_i = i * _NB - i * (i - 1) // 2
    j = t - off_i + i

    @pl.when(t == 0)
    def _prologue():
        e = e_ref[...]
        oh = jnp.where(
            lab_ref[...] == jax.lax.broadcasted_iota(
                jnp.int32, (_N, _NUM_CLASSES), 1),
            jnp.float32(128.0), jnp.float32(0.0))
        sq = jnp.sum(e * e, axis=1, keepdims=True)
        hi = sq.astype(jnp.bfloat16).astype(f)
        lo = sq - hi
        ones = jnp.ones((_N, 1), f)
        zpad = jnp.zeros((_N, _KPAD - _KAUG), f)
        lhs_s[:, 0:_D] = (-2.0 * e).astype(jnp.bfloat16)
        rhs_s[:, 0:_D] = e.astype(jnp.bfloat16)
        lhs_s[:, _D:_KPAD] = jnp.concatenate(
            [hi, lo, ones, ones, oh, zpad], axis=1).astype(jnp.bfloat16)
        rhs_s[:, _D:_KPAD] = jnp.concatenate(
            [ones, ones, hi, lo, oh, zpad], axis=1).astype(jnp.bfloat16)
        rp_s[...] = jnp.full((_N, _LANES), -jnp.inf, f)
        rn_s[...] = jnp.full((_N, _LANES), jnp.inf, f)
        cp_s[...] = jnp.full((_NB, _BLK), -jnp.inf, f)
        cn_s[...] = jnp.full((_NB, _BLK), jnp.inf, f)

    dims = (((1,), (1,)), ((), ()))
    lhs = lhs_s[pl.ds(i * _BLK, _BLK), :]
    rhs = rhs_s[pl.ds(j * _BLK, _BLK), :]
    tb = jax.lax.dot_general(lhs, rhs, dims, preferred_element_type=f)

    qs = [tb[:, g * _LANES:(g + 1) * _LANES] for g in range(_BLK // _LANES)]
    rmax = functools.reduce(jnp.maximum, qs)
    rmin = functools.reduce(jnp.minimum, qs)
    rsl = pl.ds(i * _BLK, _BLK)
    rp_s[rsl, :] = jnp.maximum(rp_s[rsl, :], rmax)
    rn_s[rsl, :] = jnp.minimum(rn_s[rsl, :], rmin)

    col_max = jnp.max(tb, axis=0, keepdims=True)
    col_min = jnp.min(tb, axis=0, keepdims=True)
    cp_s[pl.ds(j, 1), :] = jnp.maximum(cp_s[pl.ds(j, 1), :], col_max)
    cn_s[pl.ds(j, 1), :] = jnp.minimum(cn_s[pl.ds(j, 1), :], col_min)

    @pl.when(t == _STEPS - 1)
    def _epilogue():
        pos_rows = []
        neg_rows = []
        for b in range(_NB):
            bsl = pl.ds(b * _BLK, _BLK)
            pb = jnp.max(rp_s[bsl, :], axis=1, keepdims=True)
            nb_ = jnp.min(rn_s[bsl, :], axis=1, keepdims=True)
            pos_rows.append(pb.reshape(1, _BLK))
            neg_rows.append(nb_.reshape(1, _BLK))
        pos = jnp.maximum(jnp.concatenate(pos_rows, axis=0), cp_s[...])
        neg = jnp.minimum(jnp.concatenate(neg_rows, axis=0), cn_s[...])
        out_ref[...] = jnp.sum(
            jnp.maximum(pos - _TWO_B - neg + _MARGIN, 0.0), keepdims=True
        ).reshape(1, 1)


def kernel(embeds, labels):
    total = pl.pallas_call(
        _triplet_kernel,
        grid=(_STEPS,),
        in_specs=[
            pl.BlockSpec((_N, _D), lambda t: (0, 0)),
            pl.BlockSpec((_N, 1), lambda t: (0, 0)),
        ],
        out_specs=pl.BlockSpec((1, 1), lambda t: (0, 0)),
        out_shape=jax.ShapeDtypeStruct((1, 1), jnp.float32),
        scratch_shapes=[
            pltpu.VMEM((_N, _KPAD), jnp.bfloat16),
            pltpu.VMEM((_N, _KPAD), jnp.bfloat16),
            pltpu.VMEM((_N, _LANES), jnp.float32),
            pltpu.VMEM((_N, _LANES), jnp.float32),
            pltpu.VMEM((_NB, _BLK), jnp.float32),
            pltpu.VMEM((_NB, _BLK), jnp.float32),
        ],
        compiler_params=pltpu.CompilerParams(
            dimension_semantics=("arbitrary",),
        ),
    )(embeds, labels.reshape(_N, 1))

    return total[0, 0] / _N
